# baseline (speedup 1.0000x reference)
import functools

import jax
import jax.numpy as jnp
from jax.experimental import pallas as pl
from jax.experimental.pallas import tpu as pltpu

N = 4096
D = 2048
H = 256
HEAD = 1024
CPAD = 128
BLK = 512
NI = N // BLK
THRESH = 0.05

_PREC = jax.lax.Precision.HIGHEST


def _dot(a, b, precision=_PREC):
    return jax.lax.dot_general(a, b, (((1,), (0,)), ((), ())),
                               precision=precision)


def _gelu(v):
    return 0.5 * v * (1.0 + jax.lax.erf(v * (2.0 ** -0.5)))


def _layer_norm(v, g, b, eps=1e-5):
    mu = jnp.mean(v, axis=-1, keepdims=True)
    var = jnp.mean((v - mu) ** 2, axis=-1, keepdims=True)
    return (v - mu) * jax.lax.rsqrt(var + eps) * g + b


def _graph_kernel(xi_ref, xj_ref, w0_ref, adj_ref, deg_ref, p0_ref):
    j = pl.program_id(1)
    xi = xi_ref[...]
    xj = xj_ref[...]
    ni = jax.lax.rsqrt(jnp.sum(xi * xi, axis=1, keepdims=True))
    nj = jax.lax.rsqrt(jnp.sum(xj * xj, axis=1, keepdims=True))
    xjs = xj * nj
    sim = jax.lax.dot_general(xi, xjs, (((1,), (1,)), ((), ())),
                              precision=_PREC)
    sim = sim * ni
    mask = (sim >= THRESH).astype(jnp.float32)
    adj_ref[...] = mask.astype(jnp.int8)
    rs = jnp.broadcast_to(jnp.sum(mask, axis=1, keepdims=True), (BLK, 128))

    @pl.when(j == 0)
    def _():
        deg_ref[...] = rs
        p0_ref[...] = _dot(xi, w0_ref[...])

    @pl.when(j != 0)
    def _():
        deg_ref[...] += rs


def _build_graph(x, W0):
    adj, deg, p0 = pl.pallas_call(
        _graph_kernel,
        grid=(NI, NI),
        in_specs=[
            pl.BlockSpec((BLK, D), lambda i, j: (i, 0)),
            pl.BlockSpec((BLK, D), lambda i, j: (j, 0)),
            pl.BlockSpec((D, H), lambda i, j: (0, 0)),
        ],
        out_specs=[
            pl.BlockSpec((BLK, BLK), lambda i, j: (i, j)),
            pl.BlockSpec((BLK, 128), lambda i, j: (i, 0)),
            pl.BlockSpec((BLK, H), lambda i, j: (i, 0)),
        ],
        out_shape=[
            jax.ShapeDtypeStruct((N, N), jnp.int8),
            jax.ShapeDtypeStruct((N, 128), jnp.float32),
            jax.ShapeDtypeStruct((N, H), jnp.float32),
        ],
    )(x, x, W0)
    return adj, deg, p0


def _gcn_kernel(adj_ref, deg_ref, p_ref, b_ref, lng_ref, lnb_ref, w_ref,
                out_ref, cnt_ref, *, last, first):
    i = pl.program_id(0)
    deg = deg_ref[...][:, 0:1]
    dinv = jax.lax.rsqrt(deg)
    ps = p_ref[...] * dinv
    adjf = adj_ref[...].astype(jnp.float32)
    agg = _dot(adjf, ps)
    dinv_i = jax.lax.rsqrt(deg_ref[pl.ds(i * BLK, BLK), 0:1])
    agg = agg * dinv_i + b_ref[...]
    if last:
        out_ref[...] = agg
    else:
        h = _gelu(_layer_norm(agg, lng_ref[...], lnb_ref[...]))
        out_ref[...] = _dot(h, w_ref[...])
    if first:
        @pl.when(i == 0)
        def _():
            cnt_ref[0, 0] = jnp.sum(deg[:, 0])


def _gcn_layer(adj, deg, p, b, lng, lnb, w, *, last=False, first=False):
    body = functools.partial(_gcn_kernel, last=last, first=first)
    in_specs = [
        pl.BlockSpec((BLK, N), lambda i: (i, 0)),
        pl.BlockSpec((N, 128), lambda i: (0, 0)),
        pl.BlockSpec((N, H), lambda i: (0, 0)),
        pl.BlockSpec((1, H), lambda i: (0, 0)),
        pl.BlockSpec((1, H), lambda i: (0, 0)),
        pl.BlockSpec((1, H), lambda i: (0, 0)),
        pl.BlockSpec((H, H), lambda i: (0, 0)),
    ]
    out_specs = [pl.BlockSpec((BLK, H), lambda i: (i, 0)),
                 pl.BlockSpec(memory_space=pltpu.SMEM)]
    out_shape = [jax.ShapeDtypeStruct((N, H), jnp.float32),
                 jax.ShapeDtypeStruct((1, 1), jnp.float32)]
    out, cnt = pl.pallas_call(
        body,
        grid=(NI,),
        in_specs=in_specs,
        out_specs=out_specs,
        out_shape=out_shape,
    )(adj, deg, p, b, lng, lnb, w)
    return out, cnt


def _head_kernel(x_ref, h3_ref, hw1_ref, hb1_ref, g_ref, b_ref, hw2_ref,
                 hb2_ref, sw_ref, sb_ref, lm_ref, ls_ref):
    x = x_ref[...]
    z = (_dot(x, hw1_ref[0:D, :]) + _dot(h3_ref[...], hw1_ref[D:D + H, :])
         + hb1_ref[...])
    z = _gelu(z)
    z = _layer_norm(z, g_ref[...], b_ref[...])
    lm_ref[...] = _dot(z, hw2_ref[...]) + hb2_ref[...]
    ls_ref[...] = _dot(x, sw_ref[...]) + sb_ref[...]


def _heads(x, h3, hW1, hb1, g, b, hW2p, hb2p, sWp, sbp):
    return pl.pallas_call(
        _head_kernel,
        grid=(NI,),
        in_specs=[
            pl.BlockSpec((BLK, D), lambda i: (i, 0)),
            pl.BlockSpec((BLK, H), lambda i: (i, 0)),
            pl.BlockSpec((D + H, HEAD), lambda i: (0, 0)),
            pl.BlockSpec((1, HEAD), lambda i: (0, 0)),
            pl.BlockSpec((1, HEAD), lambda i: (0, 0)),
            pl.BlockSpec((1, HEAD), lambda i: (0, 0)),
            pl.BlockSpec((HEAD, CPAD), lambda i: (0, 0)),
            pl.BlockSpec((1, CPAD), lambda i: (0, 0)),
            pl.BlockSpec((D, CPAD), lambda i: (0, 0)),
            pl.BlockSpec((1, CPAD), lambda i: (0, 0)),
        ],
        out_specs=[pl.BlockSpec((BLK, CPAD), lambda i: (i, 0)),
                   pl.BlockSpec((BLK, CPAD), lambda i: (i, 0))],
        out_shape=[jax.ShapeDtypeStruct((N, CPAD), jnp.float32),
                   jax.ShapeDtypeStruct((N, CPAD), jnp.float32)],
    )(x, h3, hW1, hb1, g, b, hW2p, hb2p, sWp, sbp)


def kernel(x, W0, b0, Wh, bh, ln1_g, ln1_b, ln2_g, ln2_b, hW1, hb1,
           hln_g, hln_b, hW2, hb2, sW, sb):
    r = lambda v: v.reshape(1, -1)
    padc = lambda m: jnp.pad(m, ((0, 0), (0, CPAD - m.shape[1])))

    adj, deg, p0 = _build_graph(x, W0)

    p1, cnt = _gcn_layer(adj, deg, p0, r(b0), r(ln1_g), r(ln1_b), Wh,
                         first=True)
    p2, _ = _gcn_layer(adj, deg, p1, r(bh), r(ln2_g), r(ln2_b), Wh)
    h3, _ = _gcn_layer(adj, deg, p2, r(bh), r(ln2_g), r(ln2_b), Wh, last=True)

    lm, ls = _heads(x, h3, hW1, r(hb1), r(hln_g), r(hln_b), padc(hW2),
                    padc(r(hb2)), padc(sW), padc(r(sb)))

    logits_main = lm[:, :100]
    logits_second = ls[:, :100]
    density = (cnt[0, 0] * (1.0 / (N * N))).astype(jnp.float32)
    return (logits_main, logits_second, density)


# speedup vs baseline: 4.4176x; 4.4176x over previous
import functools

import jax
import jax.numpy as jnp
from jax.experimental import pallas as pl
from jax.experimental.pallas import tpu as pltpu

N = 4096
D = 2048
H = 256
HEAD = 1024
CPAD = 128
BLK = 512
NI = N // BLK
THRESH = 0.05

_BF = jnp.bfloat16


def _dot(a, b):
    return jax.lax.dot_general(
        a.astype(_BF), b.astype(_BF), (((1,), (0,)), ((), ())),
        preferred_element_type=jnp.float32)


def _gelu(v):
    return 0.5 * v * (1.0 + jax.lax.erf(v * (2.0 ** -0.5)))


def _layer_norm(v, g, b, eps=1e-5):
    mu = jnp.mean(v, axis=-1, keepdims=True)
    var = jnp.mean((v - mu) ** 2, axis=-1, keepdims=True)
    return (v - mu) * jax.lax.rsqrt(var + eps) * g + b


def _prep_kernel(x_ref, w0_ref, fn_ref, p0_ref):
    x = x_ref[...]
    ninv = jax.lax.rsqrt(jnp.sum(x * x, axis=1, keepdims=True))
    fn_ref[...] = (x * ninv).astype(_BF)
    p0_ref[...] = _dot(x, w0_ref[...])


def _prep(x, W0):
    return pl.pallas_call(
        _prep_kernel,
        grid=(NI,),
        in_specs=[
            pl.BlockSpec((BLK, D), lambda i: (i, 0)),
            pl.BlockSpec((D, H), lambda i: (0, 0)),
        ],
        out_specs=[
            pl.BlockSpec((BLK, D), lambda i: (i, 0)),
            pl.BlockSpec((BLK, H), lambda i: (i, 0)),
        ],
        out_shape=[
            jax.ShapeDtypeStruct((N, D), _BF),
            jax.ShapeDtypeStruct((N, H), jnp.float32),
        ],
    )(x, W0)


def _graph_kernel(fni_ref, fna_ref, adj_ref, deg_ref):
    sim = jax.lax.dot_general(fni_ref[...], fna_ref[...],
                              (((1,), (1,)), ((), ())),
                              preferred_element_type=jnp.float32)
    mask = (sim >= THRESH).astype(jnp.float32)
    adj_ref[...] = mask.astype(jnp.int8)
    deg_ref[...] = jnp.broadcast_to(
        jnp.sum(mask, axis=1, keepdims=True), (BLK, 128))


def _build_graph(fn):
    return pl.pallas_call(
        _graph_kernel,
        grid=(NI,),
        in_specs=[
            pl.BlockSpec((BLK, D), lambda i: (i, 0)),
            pl.BlockSpec((N, D), lambda i: (0, 0)),
        ],
        out_specs=[
            pl.BlockSpec((BLK, N), lambda i: (i, 0)),
            pl.BlockSpec((BLK, 128), lambda i: (i, 0)),
        ],
        out_shape=[
            jax.ShapeDtypeStruct((N, N), jnp.int8),
            jax.ShapeDtypeStruct((N, 128), jnp.float32),
        ],
    )(fn, fn)


def _gcn_kernel(adj_ref, deg_ref, p_ref, b_ref, lng_ref, lnb_ref, w_ref,
                out_ref, cnt_ref, *, last, first):
    i = pl.program_id(0)
    deg = deg_ref[...][:, 0:1]
    dinv = jax.lax.rsqrt(deg)
    ps = p_ref[...] * dinv
    agg = _dot(adj_ref[...], ps)
    dinv_i = jax.lax.rsqrt(deg_ref[pl.ds(i * BLK, BLK), 0:1])
    agg = agg * dinv_i + b_ref[...]
    if last:
        out_ref[...] = agg
    else:
        h = _gelu(_layer_norm(agg, lng_ref[...], lnb_ref[...]))
        out_ref[...] = _dot(h, w_ref[...])
    if first:
        @pl.when(i == 0)
        def _():
            cnt_ref[0, 0] = jnp.sum(deg[:, 0])


def _gcn_layer(adj, deg, p, b, lng, lnb, w, *, last=False, first=False):
    body = functools.partial(_gcn_kernel, last=last, first=first)
    in_specs = [
        pl.BlockSpec((BLK, N), lambda i: (i, 0)),
        pl.BlockSpec((N, 128), lambda i: (0, 0)),
        pl.BlockSpec((N, H), lambda i: (0, 0)),
        pl.BlockSpec((1, H), lambda i: (0, 0)),
        pl.BlockSpec((1, H), lambda i: (0, 0)),
        pl.BlockSpec((1, H), lambda i: (0, 0)),
        pl.BlockSpec((H, H), lambda i: (0, 0)),
    ]
    out_specs = [pl.BlockSpec((BLK, H), lambda i: (i, 0)),
                 pl.BlockSpec(memory_space=pltpu.SMEM)]
    out_shape = [jax.ShapeDtypeStruct((N, H), jnp.float32),
                 jax.ShapeDtypeStruct((1, 1), jnp.float32)]
    out, cnt = pl.pallas_call(
        body,
        grid=(NI,),
        in_specs=in_specs,
        out_specs=out_specs,
        out_shape=out_shape,
    )(adj, deg, p, b, lng, lnb, w)
    return out, cnt


def _head_kernel(x_ref, h3_ref, hw1_ref, hb1_ref, g_ref, b_ref, hw2_ref,
                 hb2_ref, sw_ref, sb_ref, lm_ref, ls_ref):
    x = x_ref[...]
    z = (_dot(x, hw1_ref[0:D, :]) + _dot(h3_ref[...], hw1_ref[D:D + H, :])
         + hb1_ref[...])
    z = _gelu(z)
    z = _layer_norm(z, g_ref[...], b_ref[...])
    lm_ref[...] = _dot(z, hw2_ref[...]) + hb2_ref[...]
    ls_ref[...] = _dot(x, sw_ref[...]) + sb_ref[...]


def _heads(x, h3, hW1, hb1, g, b, hW2p, hb2p, sWp, sbp):
    return pl.pallas_call(
        _head_kernel,
        grid=(NI,),
        in_specs=[
            pl.BlockSpec((BLK, D), lambda i: (i, 0)),
            pl.BlockSpec((BLK, H), lambda i: (i, 0)),
            pl.BlockSpec((D + H, HEAD), lambda i: (0, 0)),
            pl.BlockSpec((1, HEAD), lambda i: (0, 0)),
            pl.BlockSpec((1, HEAD), lambda i: (0, 0)),
            pl.BlockSpec((1, HEAD), lambda i: (0, 0)),
            pl.BlockSpec((HEAD, CPAD), lambda i: (0, 0)),
            pl.BlockSpec((1, CPAD), lambda i: (0, 0)),
            pl.BlockSpec((D, CPAD), lambda i: (0, 0)),
            pl.BlockSpec((1, CPAD), lambda i: (0, 0)),
        ],
        out_specs=[pl.BlockSpec((BLK, CPAD), lambda i: (i, 0)),
                   pl.BlockSpec((BLK, CPAD), lambda i: (i, 0))],
        out_shape=[jax.ShapeDtypeStruct((N, CPAD), jnp.float32),
                   jax.ShapeDtypeStruct((N, CPAD), jnp.float32)],
    )(x, h3, hW1, hb1, g, b, hW2p, hb2p, sWp, sbp)


def kernel(x, W0, b0, Wh, bh, ln1_g, ln1_b, ln2_g, ln2_b, hW1, hb1,
           hln_g, hln_b, hW2, hb2, sW, sb):
    r = lambda v: v.reshape(1, -1)
    padc = lambda m: jnp.pad(m, ((0, 0), (0, CPAD - m.shape[1])))

    fn, p0 = _prep(x, W0)
    adj, deg = _build_graph(fn)

    p1, cnt = _gcn_layer(adj, deg, p0, r(b0), r(ln1_g), r(ln1_b), Wh,
                         first=True)
    p2, _ = _gcn_layer(adj, deg, p1, r(bh), r(ln2_g), r(ln2_b), Wh)
    h3, _ = _gcn_layer(adj, deg, p2, r(bh), r(ln2_g), r(ln2_b), Wh, last=True)

    lm, ls = _heads(x, h3, hW1, r(hb1), r(hln_g), r(hln_b), padc(hW2),
                    padc(r(hb2)), padc(sW), padc(r(sb)))

    logits_main = lm[:, :100]
    logits_second = ls[:, :100]
    density = (cnt[0, 0] * (1.0 / (N * N))).astype(jnp.float32)
    return (logits_main, logits_second, density)


# speedup vs baseline: 4.7372x; 1.0723x over previous
import jax
import jax.numpy as jnp
from jax.experimental import pallas as pl
from jax.experimental.pallas import tpu as pltpu

N = 4096
D = 2048
H = 256
HEAD = 1024
CPAD = 128
BLK = 512
NI = N // BLK
CHUNK = 1024
THRESH = 0.05

_BF = jnp.bfloat16


def _dot(a, b):
    return jax.lax.dot_general(
        a.astype(_BF), b.astype(_BF), (((1,), (0,)), ((), ())),
        preferred_element_type=jnp.float32)


def _gelu(v):
    return 0.5 * v * (1.0 + jax.lax.erf(v * (2.0 ** -0.5)))


def _layer_norm(v, g, b, eps=1e-5):
    mu = jnp.mean(v, axis=-1, keepdims=True)
    var = jnp.mean((v - mu) ** 2, axis=-1, keepdims=True)
    return (v - mu) * jax.lax.rsqrt(var + eps) * g + b


def _graph_kernel(x_ref, w0_ref, p0_ref, adj_ref, deg_ref, fn_ref):
    t = pl.program_id(0)

    @pl.when(t < NI)
    def _prep():
        x = x_ref[...]
        ninv = jax.lax.rsqrt(jnp.sum(x * x, axis=1, keepdims=True))
        fn_ref[pl.ds(t * BLK, BLK), :] = (x * ninv).astype(_BF)
        p0_ref[...] = _dot(x, w0_ref[...])

    @pl.when(t >= NI)
    def _sim():
        i = t - NI
        fni = fn_ref[pl.ds(i * BLK, BLK), :]
        degv = jnp.zeros((BLK, 1), jnp.float32)
        for c in range(N // CHUNK):
            fnc = fn_ref[pl.ds(c * CHUNK, CHUNK), :]
            simc = jax.lax.dot_general(fni, fnc, (((1,), (1,)), ((), ())),
                                       preferred_element_type=jnp.float32)
            maskc = (simc >= THRESH).astype(jnp.float32)
            adj_ref[:, pl.ds(c * CHUNK, CHUNK)] = maskc.astype(jnp.int8)
            degv = degv + jnp.sum(maskc, axis=1, keepdims=True)
        deg_ref[...] = jnp.broadcast_to(degv, (BLK, 128))


def _build_graph(x, W0):
    return pl.pallas_call(
        _graph_kernel,
        grid=(2 * NI,),
        in_specs=[
            pl.BlockSpec((BLK, D), lambda t: (jnp.minimum(t, NI - 1), 0)),
            pl.BlockSpec((D, H), lambda t: (0, 0)),
        ],
        out_specs=[
            pl.BlockSpec((BLK, H), lambda t: (jnp.minimum(t, NI - 1), 0)),
            pl.BlockSpec((BLK, N), lambda t: (jnp.maximum(t - NI, 0), 0)),
            pl.BlockSpec((BLK, 128), lambda t: (jnp.maximum(t - NI, 0), 0)),
        ],
        out_shape=[
            jax.ShapeDtypeStruct((N, H), jnp.float32),
            jax.ShapeDtypeStruct((N, N), jnp.int8),
            jax.ShapeDtypeStruct((N, 128), jnp.float32),
        ],
        scratch_shapes=[pltpu.VMEM((N, D), _BF)],
    )(x, W0)


def _gcn_kernel(adj_ref, deg_ref, p_ref, b0_ref, bh_ref, ln1g_ref, ln1b_ref,
                ln2g_ref, ln2b_ref, wh_ref, h3_ref, cnt_ref, s0_ref, s1_ref):
    t = pl.program_id(0)
    i = t % NI

    @pl.when(t == 0)
    def _():
        cnt_ref[0, 0] = jnp.sum(deg_ref[...][:, 0])

    dinv = jax.lax.rsqrt(deg_ref[...][:, 0:1])
    dinv_i = jax.lax.rsqrt(deg_ref[pl.ds(i * BLK, BLK), 0:1])

    def agg_of(p_val):
        return _dot(adj_ref[...], p_val * dinv) * dinv_i

    @pl.when(t < NI)
    def _layer0():
        a = agg_of(p_ref[...]) + b0_ref[...]
        h = _gelu(_layer_norm(a, ln1g_ref[...], ln1b_ref[...]))
        s0_ref[pl.ds(i * BLK, BLK), :] = _dot(h, wh_ref[...])

    @pl.when((t >= NI) & (t < 2 * NI))
    def _layer1():
        a = agg_of(s0_ref[...]) + bh_ref[...]
        h = _gelu(_layer_norm(a, ln2g_ref[...], ln2b_ref[...]))
        s1_ref[pl.ds(i * BLK, BLK), :] = _dot(h, wh_ref[...])

    @pl.when(t >= 2 * NI)
    def _layer2():
        h3_ref[...] = agg_of(s1_ref[...]) + bh_ref[...]


def _gcn(adj, deg, p0, b0, bh, ln1g, ln1b, ln2g, ln2b, Wh):
    cvec = lambda: pl.BlockSpec((1, H), lambda t: (0, 0))
    h3, cnt = pl.pallas_call(
        _gcn_kernel,
        grid=(3 * NI,),
        in_specs=[
            pl.BlockSpec((BLK, N), lambda t: (t % NI, 0)),
            pl.BlockSpec((N, 128), lambda t: (0, 0)),
            pl.BlockSpec((N, H), lambda t: (0, 0)),
            cvec(), cvec(), cvec(), cvec(), cvec(), cvec(),
            pl.BlockSpec((H, H), lambda t: (0, 0)),
        ],
        out_specs=[
            pl.BlockSpec((BLK, H), lambda t: (jnp.maximum(t - 2 * NI, 0), 0)),
            pl.BlockSpec(memory_space=pltpu.SMEM),
        ],
        out_shape=[
            jax.ShapeDtypeStruct((N, H), jnp.float32),
            jax.ShapeDtypeStruct((1, 1), jnp.float32),
        ],
        scratch_shapes=[pltpu.VMEM((N, H), jnp.float32),
                        pltpu.VMEM((N, H), jnp.float32)],
    )(adj, deg, p0, b0, bh, ln1g, ln1b, ln2g, ln2b, Wh)
    return h3, cnt


def _head_kernel(x_ref, h3_ref, hw1_ref, hb1_ref, g_ref, b_ref, hw2_ref,
                 hb2_ref, sw_ref, sb_ref, lm_ref, ls_ref):
    x = x_ref[...]
    z = (_dot(x, hw1_ref[0:D, :]) + _dot(h3_ref[...], hw1_ref[D:D + H, :])
         + hb1_ref[...])
    z = _gelu(z)
    z = _layer_norm(z, g_ref[...], b_ref[...])
    lm_ref[...] = _dot(z, hw2_ref[...]) + hb2_ref[...]
    ls_ref[...] = _dot(x, sw_ref[...]) + sb_ref[...]


def _heads(x, h3, hW1, hb1, g, b, hW2p, hb2p, sWp, sbp):
    return pl.pallas_call(
        _head_kernel,
        grid=(NI,),
        in_specs=[
            pl.BlockSpec((BLK, D), lambda i: (i, 0)),
            pl.BlockSpec((BLK, H), lambda i: (i, 0)),
            pl.BlockSpec((D + H, HEAD), lambda i: (0, 0)),
            pl.BlockSpec((1, HEAD), lambda i: (0, 0)),
            pl.BlockSpec((1, HEAD), lambda i: (0, 0)),
            pl.BlockSpec((1, HEAD), lambda i: (0, 0)),
            pl.BlockSpec((HEAD, CPAD), lambda i: (0, 0)),
            pl.BlockSpec((1, CPAD), lambda i: (0, 0)),
            pl.BlockSpec((D, CPAD), lambda i: (0, 0)),
            pl.BlockSpec((1, CPAD), lambda i: (0, 0)),
        ],
        out_specs=[pl.BlockSpec((BLK, CPAD), lambda i: (i, 0)),
                   pl.BlockSpec((BLK, CPAD), lambda i: (i, 0))],
        out_shape=[jax.ShapeDtypeStruct((N, CPAD), jnp.float32),
                   jax.ShapeDtypeStruct((N, CPAD), jnp.float32)],
    )(x, h3, hW1, hb1, g, b, hW2p, hb2p, sWp, sbp)


def kernel(x, W0, b0, Wh, bh, ln1_g, ln1_b, ln2_g, ln2_b, hW1, hb1,
           hln_g, hln_b, hW2, hb2, sW, sb):
    r = lambda v: v.reshape(1, -1)
    padc = lambda m: jnp.pad(m, ((0, 0), (0, CPAD - m.shape[1])))

    p0, adj, deg = _build_graph(x, W0)
    h3, cnt = _gcn(adj, deg, p0, r(b0), r(bh), r(ln1_g), r(ln1_b),
                   r(ln2_g), r(ln2_b), Wh)
    lm, ls = _heads(x, h3, hW1, r(hb1), r(hln_g), r(hln_b), padc(hW2),
                    padc(r(hb2)), padc(sW), padc(r(sb)))

    logits_main = lm[:, :100]
    logits_second = ls[:, :100]
    density = (cnt[0, 0] * (1.0 / (N * N))).astype(jnp.float32)
    return (logits_main, logits_second, density)
